# baseline (speedup 1.0000x reference)
import functools

import jax
import jax.numpy as jnp
from jax import lax
from jax.experimental import pallas as pl
from jax.experimental.pallas import tpu as pltpu
from jax.experimental.pallas import tpu_sc as plsc

_BINS = 32
_EMBED = 64
_B = 64
_L = 4096
_NC = 2
_NS = 16
_NW = _NC * _NS
_ROWS_PER_W = _B // _NW
_CHUNK = 128
_LANES = 16


def _body(query_hbm, key_hbm, table_hbm, out_hbm, query_v, keys_v, idx_v,
          rows_v, sem):
    wid = lax.axis_index("s") * _NC + lax.axis_index("c")
    pltpu.sync_copy(query_hbm, query_v)
    base = wid * _ROWS_PER_W
    vbase = (base // _LANES) * _LANES
    qvec = query_v[pl.ds(vbase, _LANES)]
    for r in range(_ROWS_PER_W):
        b = base + r
        lane = b - vbase
        q = qvec.at[jnp.full((_LANES,), lane, jnp.int32)].get(
            mode="promise_in_bounds")

        def chunk_body(c, _, b=b, q=q):
            l0 = c * _CHUNK

            pltpu.sync_copy(key_hbm.at[b, pl.ds(l0, _CHUNK)], keys_v)

            def vec(i, _):
                kv = keys_v[pl.ds(i * _LANES, _LANES)]
                d = jnp.clip(kv - q, -_BINS, _BINS) + (_BINS + 1)
                idx_v[pl.ds(i * _LANES, _LANES)] = d
                return 0

            lax.fori_loop(0, _CHUNK // _LANES, vec, 0)

            pltpu.async_copy(table_hbm.at[idx_v], rows_v, sem).wait()
            pltpu.sync_copy(rows_v, out_hbm.at[b, pl.ds(l0, _CHUNK)])
            return 0

        lax.fori_loop(0, _L // _CHUNK, chunk_body, 0)


@jax.jit
def kernel(query_residue_index, key_residue_index, weight):
    mesh = plsc.VectorSubcoreMesh(core_axis_name="c", subcore_axis_name="s")
    run = pl.kernel(
        _body,
        out_type=jax.ShapeDtypeStruct((_B, _L, _EMBED), jnp.float32),
        mesh=mesh,
        compiler_params=pltpu.CompilerParams(use_tc_tiling_on_sc=False),
        scratch_types=[
            pltpu.VMEM((_B,), jnp.int32),
            pltpu.VMEM((_CHUNK,), jnp.int32),
            pltpu.VMEM((_CHUNK,), jnp.int32),
            pltpu.VMEM((_CHUNK, _EMBED), jnp.float32),
            pltpu.SemaphoreType.DMA,
        ],
    )
    return run(query_residue_index, key_residue_index, weight)
